# baseline (speedup 1.0000x reference)
import functools

import jax
import jax.numpy as jnp
from jax import lax
from jax.experimental import pallas as pl
from jax.experimental.pallas import tpu as pltpu
from jax.experimental.pallas import tpu_sc as plsc

_D = 64
_LANES = 128
_ROWS_PER_CHUNK = 8
_CHUNK = _ROWS_PER_CHUNK * _LANES

_N_IDX = 4096 * 200
_N_ROWS = _N_IDX // _LANES
_NC = 2
_NS = 16
_NW = _NC * _NS
_ROWS_PER_W = _N_ROWS // _NW
_N_CHUNKS = _ROWS_PER_W // _ROWS_PER_CHUNK

_mesh = plsc.VectorSubcoreMesh(core_axis_name="c", subcore_axis_name="s")


@functools.partial(
    pl.kernel,
    mesh=_mesh,
    out_type=jax.ShapeDtypeStruct((_N_IDX, _D), jnp.float32),
    scratch_types=[
        pltpu.VMEM((_ROWS_PER_CHUNK, _LANES), jnp.int32),
        pltpu.VMEM((_CHUNK, _D), jnp.float32),
        pltpu.SemaphoreType.DMA,
    ],
    compiler_params=pltpu.CompilerParams(use_tc_tiling_on_sc=False),
)
def _sc_gather(idx_hbm, table_hbm, out_hbm, idx_v, rows_v, sem):
    wid = lax.axis_index("s") * _NC + lax.axis_index("c")
    row0 = wid * _ROWS_PER_W

    def chunk_body(i, carry):
        r = row0 + i * _ROWS_PER_CHUNK
        pltpu.sync_copy(idx_hbm.at[pl.ds(r, _ROWS_PER_CHUNK)], idx_v)
        copies = [
            pltpu.async_copy(
                table_hbm.at[idx_v.at[j]],
                rows_v.at[pl.ds(j * _LANES, _LANES)],
                sem,
            )
            for j in range(_ROWS_PER_CHUNK)
        ]
        for c in copies:
            c.wait()
        pltpu.sync_copy(rows_v, out_hbm.at[pl.ds(r * _LANES, _CHUNK)])
        return carry

    lax.fori_loop(0, _N_CHUNKS, chunk_body, 0)


def kernel(sentence, elmo_tensor, word_embed):
    del elmo_tensor
    batch, seq = sentence.shape
    idx2d = sentence.reshape(_N_ROWS, _LANES)
    out = _sc_gather(idx2d, word_embed)
    return out.reshape(batch, seq, _D)
